# baseline (speedup 1.0000x reference)
import functools

import jax
import jax.numpy as jnp
from jax import lax
from jax.experimental import pallas as pl
from jax.experimental.pallas import tpu as pltpu
from jax.experimental.pallas import tpu_sc as plsc

B = 8192
K = 26
D = 256
HD = D // 2
H = 512
NLAT = 19683

NC = 2
NS = 16
NW = NC * NS
NSPLIT = 2
BS = B // NSPLIT
CPW = BS // NW
RPW = CPW * K
PCH = 128
NCH = RPW // PCH
NBUF = 3

QCLIP = 4.0
QSCALE = 1023.0 / (2.0 * QCLIP)

BB = CPW
F32 = jnp.float32
BF16 = jnp.bfloat16
U32 = jnp.uint32


def _tables_body(lat_ref, wmbe_ref, wmbo_ref, wgb_ref, p_ref, q_ref):
    lat16 = lat_ref[...].astype(BF16)
    pe = jnp.dot(lat16, wmbe_ref[...], preferred_element_type=F32)
    po = jnp.dot(lat16, wmbo_ref[...], preferred_element_type=F32)
    peu = lax.bitcast_convert_type(pe.astype(BF16), jnp.uint16).astype(U32)
    pou = lax.bitcast_convert_type(po.astype(BF16), jnp.uint16).astype(U32)
    p_ref[...] = lax.bitcast_convert_type((pou << 16) | peu, jnp.int32)

    qf = jnp.dot(lat16, wgb_ref[...], preferred_element_type=F32)
    qc = (jnp.clip(qf, -QCLIP, QCLIP) + QCLIP) * QSCALE + 0.5
    qu = qc.astype(U32)
    packed = (qu[:, 0:1] | (qu[:, 1:2] << 10) | (qu[:, 2:3] << 20))
    q_ref[...] = lax.bitcast_convert_type(packed, jnp.int32)


def _sc_gather_body(p_hbm, qt_hbm, fidxp_hbm, fidxq_hbm, pg_hbm, qg_hbm,
                    idx_all, qidx_all, qg_st, prow0, prow1, prow2,
                    qrow0, qrow1, qrow2,
                    semi, semj, semg,
                    semp0, semp1, semp2, semq0, semq1, semq2,
                    semw0, semw1, semw2):
    wid = lax.axis_index("s") * NC + lax.axis_index("c")
    rbase = wid * RPW
    bufs = ((prow0, qrow0, semp0, semq0, semw0),
            (prow1, qrow1, semp1, semq1, semw1),
            (prow2, qrow2, semp2, semq2, semw2))

    cpi = pltpu.async_copy(fidxp_hbm.at[pl.ds(rbase, RPW)], idx_all, semi)
    cpj = pltpu.async_copy(fidxq_hbm.at[pl.ds(rbase, RPW)], qidx_all, semj)
    cpi.wait()
    cpj.wait()

    def start_gather(ch, pr, qr, sp, sq):
        isl = idx_all.at[pl.ds(ch * PCH, PCH)]
        jsl = qidx_all.at[pl.ds(ch * PCH, PCH)]
        pltpu.async_copy(p_hbm.at[isl], pr, sp)
        pltpu.async_copy(qt_hbm.at[jsl], qr, sq)

    for b in range(NBUF):
        start_gather(b, bufs[b][0], bufs[b][1], bufs[b][2], bufs[b][3])

    def chunk(ch, carry):
        for b in range(NBUF):
            pr, qr, sp, sq, sw = bufs[b]

            @pl.when(lax.rem(ch, NBUF) == b)
            def _():
                isl = idx_all.at[pl.ds(0, PCH)]
                pltpu.make_async_copy(p_hbm.at[isl], pr, sp).wait()
                pltpu.async_copy(pr, pg_hbm.at[pl.ds(rbase + ch * PCH, PCH)],
                                 sw)
                pltpu.make_async_copy(qt_hbm.at[isl], qr, sq).wait()
                for v in range(PCH // 16):
                    qg_st[pl.ds(ch * PCH + v * 16, 16)] = qr[pl.ds(v * 16, 16)]

                @pl.when(ch + NBUF < NCH)
                def _():
                    pltpu.make_async_copy(
                        pr, pg_hbm.at[pl.ds(rbase, PCH)], sw).wait()
                    start_gather(ch + NBUF, pr, qr, sp, sq)
        return carry

    lax.fori_loop(0, NCH, chunk, 0)
    for b in range(NBUF):
        pr, _, _, _, sw = bufs[b]
        pltpu.make_async_copy(pr, pg_hbm.at[pl.ds(rbase, PCH)], sw).wait()
    pltpu.async_copy(qg_st, qg_hbm.at[pl.ds(rbase, RPW)], semg).wait()


def _moe_body(cs_ref, pg_ref, qg_ref, wmtp_ref, wl_ref, wut_ref, wubp_ref,
              wc1_ref, wc2_ref, wg8_ref, bmsgp_ref, bl_ref, bupd_ref,
              bc1_ref, bc2_ref, bg8_ref, out_ref):
    cs = cs_ref[...]
    cs16 = cs.astype(BF16)

    ap = (jnp.dot(cs16, wmtp_ref[...], preferred_element_type=F32)
          + bmsgp_ref[...])
    ae = ap[:, :HD]
    ao = ap[:, HD:]
    acce = jnp.zeros_like(ae)
    acco = jnp.zeros_like(ao)
    for k in range(K):
        pk = pg_ref[k * BB:(k + 1) * BB, :]
        lo = lax.bitcast_convert_type(pk << 16, F32)
        hi = lax.bitcast_convert_type(pk & jnp.int32(-65536), F32)
        acce = acce + jnp.tanh(ae + lo)
        acco = acco + jnp.tanh(ao + hi)
    aggp = jnp.concatenate([acce, acco], axis=-1) * (1.0 / K)

    qg = qg_ref[...]
    s0 = jnp.sum((qg & 1023).astype(F32), -1, keepdims=True)
    s1 = jnp.sum(((qg >> 10) & 1023).astype(F32), -1, keepdims=True)
    s2 = jnp.sum(((qg >> 20) & 1023).astype(F32), -1, keepdims=True)
    gl = (jnp.dot(cs16, wg8_ref[...], preferred_element_type=F32)
          + bg8_ref[...])
    dq = 1.0 / (QSCALE * K)
    l0 = gl[:, 0:1] + s0 * dq - QCLIP
    l1 = gl[:, 1:2] + s1 * dq - QCLIP
    l2 = gl[:, 2:3] + s2 * dq - QCLIP
    m = jnp.maximum(jnp.maximum(l0, l1), l2)
    e0 = jnp.exp(l0 - m)
    e1 = jnp.exp(l1 - m)
    e2 = jnp.exp(l2 - m)
    esum = e0 + e1 + e2

    local = jnp.tanh(jnp.dot(cs16, wl_ref[...], preferred_element_type=F32)
                     + bl_ref[...])
    func = jnp.tanh(jnp.dot(cs16, wut_ref[...], preferred_element_type=F32)
                    + jnp.dot(aggp.astype(BF16), wubp_ref[...],
                              preferred_element_type=F32)
                    + bupd_ref[...])

    x = cs
    for _ in range(3):
        h = jnp.tanh(jnp.dot(x.astype(BF16), wc1_ref[...],
                             preferred_element_type=F32) + bc1_ref[...])
        dx = jnp.dot(h.astype(BF16), wc2_ref[...],
                     preferred_element_type=F32) + bc2_ref[...]
        x = x + jnp.float32(0.1) * dx

    out_ref[...] = (e0 * local + e1 * func + e2 * x) / esum


def kernel(current_state, cell_idx, neighbor_indices, full_lattice_states,
           W_g, b_g, W_l, b_l, W_msg, b_msg, W_upd, b_upd,
           W_c1, b_c1, W_c2, b_c2):
    del cell_idx
    wmt = W_msg[:D]
    wmb = W_msg[D:]
    wmtp = jnp.concatenate([wmt[:, 0::2], wmt[:, 1::2]], 1).astype(BF16)
    bmsgp = jnp.concatenate([b_msg[0::2], b_msg[1::2]]).reshape(1, D)
    wmbe = wmb[:, 0::2].astype(BF16)
    wmbo = wmb[:, 1::2].astype(BF16)
    wg8t = jnp.pad(W_g[:D], ((0, 0), (0, 5))).astype(BF16)
    wg8b = jnp.pad(W_g[D:], ((0, 0), (0, 5))).astype(BF16)
    bg8 = jnp.pad(b_g, (0, 5)).reshape(1, 8)
    wl = W_l.astype(BF16)
    wut = W_upd[:D].astype(BF16)
    wub = W_upd[D:]
    wubp = jnp.concatenate([wub[0::2, :], wub[1::2, :]], 0).astype(BF16)
    wc1 = W_c1.astype(BF16)
    wc2 = W_c2.astype(BF16)
    bl = b_l.reshape(1, D)
    bupd = b_upd.reshape(1, D)
    bc1 = b_c1.reshape(1, H)
    bc2 = b_c2.reshape(1, D)
    nidx = neighbor_indices.astype(jnp.int32)
    fidxp = [nidx[sp * BS:(sp + 1) * BS]
             .reshape(NW, CPW, K).transpose(0, 2, 1).reshape(BS * K)
             for sp in range(NSPLIT)]
    fidxq = [nidx[sp * BS:(sp + 1) * BS].reshape(BS * K)
             for sp in range(NSPLIT)]
    cs_s = [current_state[sp * BS:(sp + 1) * BS] for sp in range(NSPLIT)]

    nblk = 1024
    ngrid = (NLAT + nblk - 1) // nblk
    p_tab, q_tab = pl.pallas_call(
        _tables_body,
        grid=(ngrid,),
        in_specs=[
            pl.BlockSpec((nblk, D), lambda i: (i, 0)),
            pl.BlockSpec((D, HD), lambda i: (0, 0)),
            pl.BlockSpec((D, HD), lambda i: (0, 0)),
            pl.BlockSpec((D, 8), lambda i: (0, 0)),
        ],
        out_specs=[
            pl.BlockSpec((nblk, HD), lambda i: (i, 0)),
            pl.BlockSpec((nblk, 1), lambda i: (i, 0)),
        ],
        out_shape=[
            jax.ShapeDtypeStruct((NLAT, HD), jnp.int32),
            jax.ShapeDtypeStruct((NLAT, 1), jnp.int32),
        ],
    )(full_lattice_states, wmbe, wmbo, wg8b)
    qt_flat = q_tab.reshape(NLAT)

    mesh = plsc.VectorSubcoreMesh(core_axis_name="c", subcore_axis_name="s")
    sc_gather = functools.partial(
        pl.kernel, mesh=mesh,
        out_type=[
            jax.ShapeDtypeStruct((BS * K, HD), jnp.int32),
            jax.ShapeDtypeStruct((BS * K,), jnp.int32),
        ],
        scratch_types=(
            [pltpu.VMEM((RPW,), jnp.int32),
             pltpu.VMEM((RPW,), jnp.int32),
             pltpu.VMEM((RPW,), jnp.int32)]
            + [pltpu.VMEM((PCH, HD), jnp.int32) for _ in range(NBUF)]
            + [pltpu.VMEM((PCH,), jnp.int32) for _ in range(NBUF)]
            + [pltpu.SemaphoreType.DMA for _ in range(3 + 3 * NBUF)]
        ),
    )(_sc_gather_body)
    scout = [sc_gather(p_tab, qt_flat, fidxp[sp], fidxq[sp])
             for sp in range(NSPLIT)]
    outs = [_moe_call(cs_s[sp], scout[sp][0], scout[sp][1],
                      (wmtp, wl, wut, wubp, wc1, wc2, wg8t, bmsgp, bl, bupd,
                       bc1, bc2, bg8))
            for sp in range(NSPLIT)]
    return jnp.concatenate(outs, axis=0)


def _moe_call(cs, pg, qg, weights):
    (wmtp, wl, wut, wubp, wc1, wc2, wg8t, bmsgp, bl, bupd, bc1, bc2,
     bg8) = weights
    qg2 = qg.reshape(BS, K)

    out = pl.pallas_call(
        _moe_body,
        grid=(BS // BB,),
        in_specs=[
            pl.BlockSpec((BB, D), lambda i: (i, 0)),
            pl.BlockSpec((BB * K, HD), lambda i: (i, 0)),
            pl.BlockSpec((BB, K), lambda i: (i, 0)),
            pl.BlockSpec((D, D), lambda i: (0, 0)),
            pl.BlockSpec((D, D), lambda i: (0, 0)),
            pl.BlockSpec((D, D), lambda i: (0, 0)),
            pl.BlockSpec((D, D), lambda i: (0, 0)),
            pl.BlockSpec((D, H), lambda i: (0, 0)),
            pl.BlockSpec((H, D), lambda i: (0, 0)),
            pl.BlockSpec((D, 8), lambda i: (0, 0)),
            pl.BlockSpec((1, D), lambda i: (0, 0)),
            pl.BlockSpec((1, D), lambda i: (0, 0)),
            pl.BlockSpec((1, D), lambda i: (0, 0)),
            pl.BlockSpec((1, H), lambda i: (0, 0)),
            pl.BlockSpec((1, D), lambda i: (0, 0)),
            pl.BlockSpec((1, 8), lambda i: (0, 0)),
        ],
        out_specs=pl.BlockSpec((BB, D), lambda i: (i, 0)),
        out_shape=jax.ShapeDtypeStruct((BS, D), F32),
    )(cs, pg, qg2, wmtp, wl, wut, wubp, wc1, wc2, wg8t,
      bmsgp, bl, bupd, bc1, bc2, bg8)
    return out


# speedup vs baseline: 1.1893x; 1.1893x over previous
import functools

import jax
import jax.numpy as jnp
from jax import lax
from jax.experimental import pallas as pl
from jax.experimental.pallas import tpu as pltpu
from jax.experimental.pallas import tpu_sc as plsc

B = 8192
K = 26
D = 256
HD = D // 2
H = 512
NLAT = 19683

NC = 2
NS = 16
NW = NC * NS
CPW = B // NW
RPW = CPW * K
PCH = 128
NCH = RPW // PCH
NBUF = 3

QCLIP = 4.0
QSCALE = 1023.0 / (2.0 * QCLIP)

BB = 256
F32 = jnp.float32
BF16 = jnp.bfloat16
U32 = jnp.uint32


def _tables_body(lat_ref, wmbe_ref, wmbo_ref, wgb_ref, p_ref, q_ref):
    lat16 = lat_ref[...].astype(BF16)
    pe = jnp.dot(lat16, wmbe_ref[...], preferred_element_type=F32)
    po = jnp.dot(lat16, wmbo_ref[...], preferred_element_type=F32)
    peu = lax.bitcast_convert_type(pe.astype(BF16), jnp.uint16).astype(U32)
    pou = lax.bitcast_convert_type(po.astype(BF16), jnp.uint16).astype(U32)
    p_ref[...] = lax.bitcast_convert_type((pou << 16) | peu, jnp.int32)

    qf = jnp.dot(lat16, wgb_ref[...], preferred_element_type=F32)
    qc = (jnp.clip(qf, -QCLIP, QCLIP) + QCLIP) * QSCALE + 0.5
    qu = qc.astype(U32)
    packed = (qu[:, 0:1] | (qu[:, 1:2] << 10) | (qu[:, 2:3] << 20))
    q_ref[...] = lax.bitcast_convert_type(packed, jnp.int32)


def _sc_gather_body(p_hbm, qt_hbm, fidxp_hbm, fidxq_hbm, pg_hbm, qg_hbm,
                    idx_all, qidx_all, qg_st, prow0, prow1, prow2,
                    qrow0, qrow1, qrow2,
                    semi, semj, semg,
                    semp0, semp1, semp2, semq0, semq1, semq2,
                    semw0, semw1, semw2):
    wid = lax.axis_index("s") * NC + lax.axis_index("c")
    rbase = wid * RPW
    bufs = ((prow0, qrow0, semp0, semq0, semw0),
            (prow1, qrow1, semp1, semq1, semw1),
            (prow2, qrow2, semp2, semq2, semw2))

    cpi = pltpu.async_copy(fidxp_hbm.at[pl.ds(rbase, RPW)], idx_all, semi)
    cpj = pltpu.async_copy(fidxq_hbm.at[pl.ds(rbase, RPW)], qidx_all, semj)
    cpi.wait()
    cpj.wait()

    def start_gather(ch, pr, qr, sp, sq):
        isl = idx_all.at[pl.ds(ch * PCH, PCH)]
        jsl = qidx_all.at[pl.ds(ch * PCH, PCH)]
        pltpu.async_copy(p_hbm.at[isl], pr, sp)
        pltpu.async_copy(qt_hbm.at[jsl], qr, sq)

    for b in range(NBUF):
        start_gather(b, bufs[b][0], bufs[b][1], bufs[b][2], bufs[b][3])

    def chunk(ch, carry):
        for b in range(NBUF):
            pr, qr, sp, sq, sw = bufs[b]

            @pl.when(lax.rem(ch, NBUF) == b)
            def _():
                isl = idx_all.at[pl.ds(0, PCH)]
                pltpu.make_async_copy(p_hbm.at[isl], pr, sp).wait()
                pltpu.async_copy(pr, pg_hbm.at[pl.ds(rbase + ch * PCH, PCH)],
                                 sw)
                pltpu.make_async_copy(qt_hbm.at[isl], qr, sq).wait()
                for v in range(PCH // 16):
                    qg_st[pl.ds(ch * PCH + v * 16, 16)] = qr[pl.ds(v * 16, 16)]

                @pl.when(ch + NBUF < NCH)
                def _():
                    pltpu.make_async_copy(
                        pr, pg_hbm.at[pl.ds(rbase, PCH)], sw).wait()
                    start_gather(ch + NBUF, pr, qr, sp, sq)
        return carry

    lax.fori_loop(0, NCH, chunk, 0)
    for b in range(NBUF):
        pr, _, _, _, sw = bufs[b]
        pltpu.make_async_copy(pr, pg_hbm.at[pl.ds(rbase, PCH)], sw).wait()
    pltpu.async_copy(qg_st, qg_hbm.at[pl.ds(rbase, RPW)], semg).wait()


def _moe_body(cs_ref, pg_ref, qg_ref, wmtp_ref, wl_ref, wut_ref, wubp_ref,
              wc1_ref, wc2_ref, wg8_ref, bmsgp_ref, bl_ref, bupd_ref,
              bc1_ref, bc2_ref, bg8_ref, out_ref):
    cs = cs_ref[...]
    cs16 = cs.astype(BF16)

    ap = (jnp.dot(cs16, wmtp_ref[...], preferred_element_type=F32)
          + bmsgp_ref[...])
    ae = ap[:, :HD]
    ao = ap[:, HD:]
    acce = jnp.zeros_like(ae)
    acco = jnp.zeros_like(ao)
    for k in range(K):
        pk = pg_ref[k * BB:(k + 1) * BB, :]
        lo = lax.bitcast_convert_type(pk << 16, F32)
        hi = lax.bitcast_convert_type(pk & jnp.int32(-65536), F32)
        acce = acce + jnp.tanh(ae + lo)
        acco = acco + jnp.tanh(ao + hi)
    aggp = jnp.concatenate([acce, acco], axis=-1) * (1.0 / K)

    qg = qg_ref[...]
    s0 = jnp.sum((qg & 1023).astype(F32), -1, keepdims=True)
    s1 = jnp.sum(((qg >> 10) & 1023).astype(F32), -1, keepdims=True)
    s2 = jnp.sum(((qg >> 20) & 1023).astype(F32), -1, keepdims=True)
    gl = (jnp.dot(cs16, wg8_ref[...], preferred_element_type=F32)
          + bg8_ref[...])
    dq = 1.0 / (QSCALE * K)
    l0 = gl[:, 0:1] + s0 * dq - QCLIP
    l1 = gl[:, 1:2] + s1 * dq - QCLIP
    l2 = gl[:, 2:3] + s2 * dq - QCLIP
    m = jnp.maximum(jnp.maximum(l0, l1), l2)
    e0 = jnp.exp(l0 - m)
    e1 = jnp.exp(l1 - m)
    e2 = jnp.exp(l2 - m)
    esum = e0 + e1 + e2

    local = jnp.tanh(jnp.dot(cs16, wl_ref[...], preferred_element_type=F32)
                     + bl_ref[...])
    func = jnp.tanh(jnp.dot(cs16, wut_ref[...], preferred_element_type=F32)
                    + jnp.dot(aggp.astype(BF16), wubp_ref[...],
                              preferred_element_type=F32)
                    + bupd_ref[...])

    x = cs
    for _ in range(3):
        h = jnp.tanh(jnp.dot(x.astype(BF16), wc1_ref[...],
                             preferred_element_type=F32) + bc1_ref[...])
        dx = jnp.dot(h.astype(BF16), wc2_ref[...],
                     preferred_element_type=F32) + bc2_ref[...]
        x = x + jnp.float32(0.1) * dx

    out_ref[...] = (e0 * local + e1 * func + e2 * x) / esum


def kernel(current_state, cell_idx, neighbor_indices, full_lattice_states,
           W_g, b_g, W_l, b_l, W_msg, b_msg, W_upd, b_upd,
           W_c1, b_c1, W_c2, b_c2):
    del cell_idx
    wmt = W_msg[:D]
    wmb = W_msg[D:]
    wmtp = jnp.concatenate([wmt[:, 0::2], wmt[:, 1::2]], 1).astype(BF16)
    bmsgp = jnp.concatenate([b_msg[0::2], b_msg[1::2]]).reshape(1, D)
    wmbe = wmb[:, 0::2].astype(BF16)
    wmbo = wmb[:, 1::2].astype(BF16)
    wg8t = jnp.pad(W_g[:D], ((0, 0), (0, 5))).astype(BF16)
    wg8b = jnp.pad(W_g[D:], ((0, 0), (0, 5))).astype(BF16)
    bg8 = jnp.pad(b_g, (0, 5)).reshape(1, 8)
    wl = W_l.astype(BF16)
    wut = W_upd[:D].astype(BF16)
    wub = W_upd[D:]
    wubp = jnp.concatenate([wub[0::2, :], wub[1::2, :]], 0).astype(BF16)
    wc1 = W_c1.astype(BF16)
    wc2 = W_c2.astype(BF16)
    bl = b_l.reshape(1, D)
    bupd = b_upd.reshape(1, D)
    bc1 = b_c1.reshape(1, H)
    bc2 = b_c2.reshape(1, D)
    nidx = neighbor_indices.astype(jnp.int32)
    fidxp = nidx.reshape(NW, CPW, K).transpose(0, 2, 1).reshape(B * K)
    fidxq = nidx.reshape(B * K)

    nblk = 1024
    ngrid = (NLAT + nblk - 1) // nblk
    p_tab, q_tab = pl.pallas_call(
        _tables_body,
        grid=(ngrid,),
        in_specs=[
            pl.BlockSpec((nblk, D), lambda i: (i, 0)),
            pl.BlockSpec((D, HD), lambda i: (0, 0)),
            pl.BlockSpec((D, HD), lambda i: (0, 0)),
            pl.BlockSpec((D, 8), lambda i: (0, 0)),
        ],
        out_specs=[
            pl.BlockSpec((nblk, HD), lambda i: (i, 0)),
            pl.BlockSpec((nblk, 1), lambda i: (i, 0)),
        ],
        out_shape=[
            jax.ShapeDtypeStruct((NLAT, HD), jnp.int32),
            jax.ShapeDtypeStruct((NLAT, 1), jnp.int32),
        ],
    )(full_lattice_states, wmbe, wmbo, wg8b)
    qt_flat = q_tab.reshape(NLAT)

    mesh = plsc.VectorSubcoreMesh(core_axis_name="c", subcore_axis_name="s")
    sc_gather = functools.partial(
        pl.kernel, mesh=mesh,
        out_type=[
            jax.ShapeDtypeStruct((B * K, HD), jnp.int32),
            jax.ShapeDtypeStruct((B * K,), jnp.int32),
        ],
        scratch_types=(
            [pltpu.VMEM((RPW,), jnp.int32),
             pltpu.VMEM((RPW,), jnp.int32),
             pltpu.VMEM((RPW,), jnp.int32)]
            + [pltpu.VMEM((PCH, HD), jnp.int32) for _ in range(NBUF)]
            + [pltpu.VMEM((PCH,), jnp.int32) for _ in range(NBUF)]
            + [pltpu.SemaphoreType.DMA for _ in range(3 + 3 * NBUF)]
        ),
    )(_sc_gather_body)
    pg, qg = sc_gather(p_tab, qt_flat, fidxp, fidxq)
    qg2 = qg.reshape(B, K)

    out = pl.pallas_call(
        _moe_body,
        grid=(B // BB,),
        in_specs=[
            pl.BlockSpec((BB, D), lambda i: (i, 0)),
            pl.BlockSpec((BB * K, HD), lambda i: (i, 0)),
            pl.BlockSpec((BB, K), lambda i: (i, 0)),
            pl.BlockSpec((D, D), lambda i: (0, 0)),
            pl.BlockSpec((D, D), lambda i: (0, 0)),
            pl.BlockSpec((D, D), lambda i: (0, 0)),
            pl.BlockSpec((D, D), lambda i: (0, 0)),
            pl.BlockSpec((D, H), lambda i: (0, 0)),
            pl.BlockSpec((H, D), lambda i: (0, 0)),
            pl.BlockSpec((D, 8), lambda i: (0, 0)),
            pl.BlockSpec((1, D), lambda i: (0, 0)),
            pl.BlockSpec((1, D), lambda i: (0, 0)),
            pl.BlockSpec((1, D), lambda i: (0, 0)),
            pl.BlockSpec((1, H), lambda i: (0, 0)),
            pl.BlockSpec((1, D), lambda i: (0, 0)),
            pl.BlockSpec((1, 8), lambda i: (0, 0)),
        ],
        out_specs=pl.BlockSpec((BB, D), lambda i: (i, 0)),
        out_shape=jax.ShapeDtypeStruct((B, D), F32),
    )(current_state, pg, qg2, wmtp, wl, wut, wubp, wc1, wc2, wg8t,
      bmsgp, bl, bupd, bc1, bc2, bg8)
    return out
